# baseline (speedup 1.0000x reference)
import jax
import jax.numpy as jnp
from jax.experimental import pallas as pl
from jax.experimental.pallas import tpu as pltpu

N_TOKENS = 16384
INPUT_DIM = 2048
REGIME_DIM = 16
N_EXPERTS = 64
HIDDEN = 128
TOP_K = 2

BLK = 512


def _router_body(x_ref, reg_ref, w1x_ref, w1r_ref, b1_ref, w2_ref, b2_ref,
                 w_out_ref, i_out_ref):
    pre = (
        jnp.dot(x_ref[...], w1x_ref[...], preferred_element_type=jnp.float32)
        + jnp.dot(reg_ref[...], w1r_ref[...], preferred_element_type=jnp.float32)
        + b1_ref[...]
    )
    h = 0.5 * pre * (1.0 + jax.lax.erf(pre * 0.7071067811865476))
    logits = jnp.dot(h, w2_ref[...], preferred_element_type=jnp.float32) + b2_ref[...]

    col = jax.lax.broadcasted_iota(jnp.int32, logits.shape, 1)
    m1 = jnp.max(logits, axis=1, keepdims=True)
    i1 = jnp.min(jnp.where(logits == m1, col, N_EXPERTS), axis=1, keepdims=True)
    masked = jnp.where(col == i1, -jnp.inf, logits)
    m2 = jnp.max(masked, axis=1, keepdims=True)
    i2 = jnp.min(jnp.where(masked == m2, col, N_EXPERTS), axis=1, keepdims=True)

    e = jnp.exp(m2 - m1)
    denom = 1.0 + e
    w_out_ref[...] = jnp.concatenate([1.0 / denom, e / denom], axis=1)
    i_out_ref[...] = jnp.concatenate([i1, i2], axis=1)


@jax.jit
def kernel(x, regime_emb, W1, b1, W2, b2):
    w1x = W1[:INPUT_DIM]
    w1r = W1[INPUT_DIM:]
    b1r = b1.reshape(1, HIDDEN)
    b2r = b2.reshape(1, N_EXPERTS)

    grid = (N_TOKENS // BLK,)
    tok = lambda i: (i, 0)
    rep = lambda i: (0, 0)
    weights, idx = pl.pallas_call(
        _router_body,
        grid=grid,
        in_specs=[
            pl.BlockSpec((BLK, INPUT_DIM), tok),
            pl.BlockSpec((BLK, REGIME_DIM), tok),
            pl.BlockSpec((INPUT_DIM, HIDDEN), rep),
            pl.BlockSpec((REGIME_DIM, HIDDEN), rep),
            pl.BlockSpec((1, HIDDEN), rep),
            pl.BlockSpec((HIDDEN, N_EXPERTS), rep),
            pl.BlockSpec((1, N_EXPERTS), rep),
        ],
        out_specs=[
            pl.BlockSpec((BLK, TOP_K), tok),
            pl.BlockSpec((BLK, TOP_K), tok),
        ],
        out_shape=[
            jax.ShapeDtypeStruct((N_TOKENS, TOP_K), jnp.float32),
            jax.ShapeDtypeStruct((N_TOKENS, TOP_K), jnp.int32),
        ],
        compiler_params=pltpu.CompilerParams(
            dimension_semantics=("arbitrary",),
        ),
    )(x, regime_emb, w1x, w1r, b1r, W2, b2r)
    return weights, idx


# speedup vs baseline: 1.2049x; 1.2049x over previous
import jax
import jax.numpy as jnp
from jax.experimental import pallas as pl
from jax.experimental.pallas import tpu as pltpu

N_TOKENS = 16384
INPUT_DIM = 2048
REGIME_DIM = 16
N_EXPERTS = 64
HIDDEN = 128
TOP_K = 2

BLK = 2048


def _router_body(x_ref, reg_ref, w1x_ref, w1r_ref, b1_ref, w2_ref, b2_ref,
                 w_out_ref, i_out_ref):
    pre = (
        jnp.dot(x_ref[...], w1x_ref[...], preferred_element_type=jnp.float32)
        + jnp.dot(reg_ref[...], w1r_ref[...], preferred_element_type=jnp.float32)
        + b1_ref[...]
    )
    h = 0.5 * pre * (1.0 + jax.lax.erf(pre * 0.7071067811865476))
    logits = jnp.dot(h, w2_ref[...], preferred_element_type=jnp.float32) + b2_ref[...]

    col = jax.lax.broadcasted_iota(jnp.int32, logits.shape, 1)
    m1 = jnp.max(logits, axis=1, keepdims=True)
    i1 = jnp.min(jnp.where(logits == m1, col, N_EXPERTS), axis=1, keepdims=True)
    masked = jnp.where(col == i1, -jnp.inf, logits)
    m2 = jnp.max(masked, axis=1, keepdims=True)
    i2 = jnp.min(jnp.where(masked == m2, col, N_EXPERTS), axis=1, keepdims=True)

    e = jnp.exp(m2 - m1)
    denom = 1.0 + e
    w_out_ref[...] = jnp.concatenate([1.0 / denom, e / denom], axis=1)
    i_out_ref[...] = jnp.concatenate([i1, i2], axis=1)


@jax.jit
def kernel(x, regime_emb, W1, b1, W2, b2):
    w1x = W1[:INPUT_DIM]
    w1r = W1[INPUT_DIM:]
    b1r = b1.reshape(1, HIDDEN)
    b2r = b2.reshape(1, N_EXPERTS)

    grid = (N_TOKENS // BLK,)
    tok = lambda i: (i, 0)
    rep = lambda i: (0, 0)
    weights, idx = pl.pallas_call(
        _router_body,
        grid=grid,
        in_specs=[
            pl.BlockSpec((BLK, INPUT_DIM), tok),
            pl.BlockSpec((BLK, REGIME_DIM), tok),
            pl.BlockSpec((INPUT_DIM, HIDDEN), rep),
            pl.BlockSpec((REGIME_DIM, HIDDEN), rep),
            pl.BlockSpec((1, HIDDEN), rep),
            pl.BlockSpec((HIDDEN, N_EXPERTS), rep),
            pl.BlockSpec((1, N_EXPERTS), rep),
        ],
        out_specs=[
            pl.BlockSpec((BLK, TOP_K), tok),
            pl.BlockSpec((BLK, TOP_K), tok),
        ],
        out_shape=[
            jax.ShapeDtypeStruct((N_TOKENS, TOP_K), jnp.float32),
            jax.ShapeDtypeStruct((N_TOKENS, TOP_K), jnp.int32),
        ],
        compiler_params=pltpu.CompilerParams(
            dimension_semantics=("arbitrary",),
        ),
    )(x, regime_emb, w1x, w1r, b1r, W2, b2r)
    return weights, idx
